# Initial kernel scaffold; baseline (speedup 1.0000x reference)
import jax
import jax.numpy as jnp
from jax.experimental import pallas as pl


def kernel(x, edge_index, W1, b1, W2, b2):
    raise NotImplementedError("write your pallas kernel here")


# speedup vs baseline: 37.9149x; 37.9149x over previous
import functools

import jax
import jax.numpy as jnp
from jax import lax
from jax.experimental import pallas as pl
from jax.experimental.pallas import tpu as pltpu
from jax.experimental.pallas import tpu_sc as plsc

N = 10000
E = 320000
D = 128
H = 16
NP = 10240
NW = 32
CH = 80
CPT = 128
EP = NW * CPT * CH

_mesh = plsc.VectorSubcoreMesh(core_axis_name="c", subcore_axis_name="s")


def _wid():
    return lax.axis_index("s") * 2 + lax.axis_index("c")


@functools.partial(
    pl.kernel, mesh=_mesh,
    compiler_params=pltpu.CompilerParams(use_tc_tiling_on_sc=False, needs_layout_passes=False),
    out_type=jax.ShapeDtypeStruct((2, NP), jnp.float32),
    scratch_types=[
        pltpu.VMEM((CPT, CH), jnp.int32),
        pltpu.VMEM((CH,), jnp.float32),
        pltpu.VMEM_SHARED((NP,), jnp.float32),
    ],
)
def _sc_hist(dst2_hbm, zeros_hbm, ones_hbm, out_hbm, dsti_v, ones_v, acc_sh):
    c = lax.axis_index("c")
    s = lax.axis_index("s")
    w = _wid()
    pltpu.sync_copy(zeros_hbm, acc_sh.at[pl.ds(s * (NP // 16), NP // 16)])
    pltpu.sync_copy(ones_hbm, ones_v)
    pltpu.sync_copy(dst2_hbm.at[pl.ds(w * CPT, CPT)], dsti_v)
    plsc.subcore_barrier()

    @pl.loop(0, CPT)
    def _(i):
        pltpu.sync_copy(ones_v, acc_sh.at[dsti_v.at[i]], add=True)

    plsc.subcore_barrier()
    pltpu.sync_copy(acc_sh.at[pl.ds(s * (NP // 16), NP // 16)],
                    out_hbm.at[c].at[pl.ds(s * (NP // 16), NP // 16)])


@functools.partial(
    pl.kernel, mesh=_mesh,
    compiler_params=pltpu.CompilerParams(use_tc_tiling_on_sc=False, needs_layout_passes=False),
    out_type=jax.ShapeDtypeStruct((2, NP, H), jnp.float32),
    scratch_types=[
        pltpu.VMEM((CPT, CH), jnp.int32),
        pltpu.VMEM((CPT, CH), jnp.int32),
        pltpu.VMEM((CH, H), jnp.float32),
        pltpu.VMEM_SHARED((NP, H), jnp.float32),
        pltpu.SemaphoreType.DMA,
    ],
)
def _sc_agg16(src2_hbm, dst2_hbm, y1_hbm, zrows_hbm, out_hbm,
              srci_v, dsti_v, rows_v, acc_sh, sem):
    c = lax.axis_index("c")
    s = lax.axis_index("s")
    w = _wid()
    pltpu.sync_copy(zrows_hbm, acc_sh.at[pl.ds(s * (NP // 16), NP // 16)])
    pltpu.sync_copy(src2_hbm.at[pl.ds(w * CPT, CPT)], srci_v)
    pltpu.sync_copy(dst2_hbm.at[pl.ds(w * CPT, CPT)], dsti_v)
    plsc.subcore_barrier()

    @pl.loop(0, CPT)
    def _(i):
        pltpu.async_copy(y1_hbm.at[srci_v.at[i]], rows_v, sem).wait()
        pltpu.sync_copy(rows_v, acc_sh.at[dsti_v.at[i]], add=True)

    plsc.subcore_barrier()
    pltpu.sync_copy(acc_sh.at[pl.ds(s * (NP // 16), NP // 16)],
                    out_hbm.at[c].at[pl.ds(s * (NP // 16), NP // 16)])


@functools.partial(
    pl.kernel, mesh=_mesh,
    compiler_params=pltpu.CompilerParams(use_tc_tiling_on_sc=False, needs_layout_passes=False),
    out_type=jax.ShapeDtypeStruct((2, NP), jnp.float32),
    scratch_types=[
        pltpu.VMEM((CPT, CH), jnp.int32),
        pltpu.VMEM((CPT, CH), jnp.int32),
        pltpu.VMEM((NP,), jnp.float32),
        pltpu.VMEM((CH,), jnp.float32),
        pltpu.VMEM_SHARED((NP,), jnp.float32),
    ],
)
def _sc_agg1(src2_hbm, dst2_hbm, y2_hbm, zeros_hbm, out_hbm,
             srci_v, dsti_v, tab_v, vals_v, acc_sh):
    c = lax.axis_index("c")
    s = lax.axis_index("s")
    w = _wid()
    pltpu.sync_copy(zeros_hbm, acc_sh.at[pl.ds(s * (NP // 16), NP // 16)])
    pltpu.sync_copy(y2_hbm, tab_v)
    pltpu.sync_copy(src2_hbm.at[pl.ds(w * CPT, CPT)], srci_v)
    pltpu.sync_copy(dst2_hbm.at[pl.ds(w * CPT, CPT)], dsti_v)
    plsc.subcore_barrier()

    @pl.loop(0, CPT)
    def _(i):
        @pl.loop(0, CH, step=16)
        def _(j):
            idx = srci_v[i, pl.ds(j, 16)]
            vals_v[pl.ds(j, 16)] = plsc.load_gather(tab_v, [idx])

        pltpu.sync_copy(vals_v, acc_sh.at[dsti_v.at[i]], add=True)

    plsc.subcore_barrier()
    pltpu.sync_copy(acc_sh.at[pl.ds(s * (NP // 16), NP // 16)],
                    out_hbm.at[c].at[pl.ds(s * (NP // 16), NP // 16)])


def _tc_mm1_body(x_ref, w_ref, o_ref):
    o_ref[...] = jnp.dot(x_ref[...], w_ref[...],
                         preferred_element_type=jnp.float32)


def _tc_scale_body(h0_ref, h1_ref, xw_ref, y1_ref, dinv_ref):
    deg = h0_ref[...] + h1_ref[...] + 1.0
    dinv = lax.rsqrt(deg)
    dinv_ref[...] = dinv
    y1_ref[...] = xw_ref[...] * dinv


def _tc_layer2_body(a0_ref, a1_ref, y1_ref, dinv_ref, b1_ref, w2_ref, o_ref):
    pre = (a0_ref[...] + a1_ref[...] + y1_ref[...]) * dinv_ref[...] + b1_ref[...]
    h = jnp.maximum(pre, 0.0)
    z = jnp.dot(h, w2_ref[...], preferred_element_type=jnp.float32)
    o_ref[0:N, :] = z * dinv_ref[...]
    o_ref[N:NP, :] = jnp.zeros((NP - N, 1), jnp.float32)


def _tc_final_body(p0_ref, p1_ref, y2_ref, dinv_ref, b2_ref, o_ref):
    o_ref[...] = dinv_ref[...] * (p0_ref[...] + p1_ref[...] + y2_ref[...]) \
        + b2_ref[...]


def kernel(x, edge_index, W1, b1, W2, b2):
    f32 = jnp.float32
    npad = EP - E
    pad_src = jnp.zeros((npad,), jnp.int32)
    pad_dst = N + (jnp.arange(npad, dtype=jnp.int32) % (NP - N))
    src2 = jnp.concatenate([edge_index[0], pad_src]).reshape(EP // CH, CH)
    dst2 = jnp.concatenate([edge_index[1], pad_dst]).reshape(EP // CH, CH)
    zeros_v = jnp.zeros((NP // 16,), f32)
    zeros_r = jnp.zeros((NP // 16, H), f32)
    ones_v = jnp.ones((CH,), f32)

    xw = pl.pallas_call(
        _tc_mm1_body,
        out_shape=jax.ShapeDtypeStruct((N, H), f32),
    )(x, W1)

    hist = _sc_hist(dst2, zeros_v, ones_v)
    h0 = hist[0, :N].reshape(N, 1)
    h1 = hist[1, :N].reshape(N, 1)

    y1, dinv = pl.pallas_call(
        _tc_scale_body,
        out_shape=(jax.ShapeDtypeStruct((N, H), f32),
                   jax.ShapeDtypeStruct((N, 1), f32)),
    )(h0, h1, xw)

    agg = _sc_agg16(src2, dst2, y1, zeros_r)

    y2p = pl.pallas_call(
        _tc_layer2_body,
        out_shape=jax.ShapeDtypeStruct((NP, 1), f32),
    )(agg[0, :N], agg[1, :N], y1, dinv, b1.reshape(1, H), W2)

    p = _sc_agg1(src2, dst2, y2p.reshape(NP), zeros_v)

    out = pl.pallas_call(
        _tc_final_body,
        out_shape=jax.ShapeDtypeStruct((N, 1), f32),
    )(p[0, :N].reshape(N, 1), p[1, :N].reshape(N, 1), y2p[:N],
      dinv, b2.reshape(1, 1))

    return out.reshape(-1)


# speedup vs baseline: 50.4434x; 1.3304x over previous
import functools

import jax
import jax.numpy as jnp
from jax import lax
from jax.experimental import pallas as pl
from jax.experimental.pallas import tpu as pltpu
from jax.experimental.pallas import tpu_sc as plsc

N = 10000
E = 320000
D = 128
H = 16
NP = 10240
NW = 32
CH = 128
CPT = 80
EP = NW * CPT * CH

_mesh = plsc.VectorSubcoreMesh(core_axis_name="c", subcore_axis_name="s")


def _wid():
    return lax.axis_index("s") * 2 + lax.axis_index("c")


@functools.partial(
    pl.kernel, mesh=_mesh,
    compiler_params=pltpu.CompilerParams(use_tc_tiling_on_sc=False, needs_layout_passes=False),
    out_type=jax.ShapeDtypeStruct((2, NP), jnp.float32),
    scratch_types=[
        pltpu.VMEM((CPT, CH), jnp.int32),
        pltpu.VMEM((CH,), jnp.float32),
        pltpu.VMEM_SHARED((NP,), jnp.float32),
    ],
)
def _sc_hist(dst2_hbm, zeros_hbm, ones_hbm, out_hbm, dsti_v, ones_v, acc_sh):
    c = lax.axis_index("c")
    s = lax.axis_index("s")
    w = _wid()
    pltpu.sync_copy(zeros_hbm, acc_sh.at[pl.ds(s * (NP // 16), NP // 16)])
    pltpu.sync_copy(ones_hbm, ones_v)
    pltpu.sync_copy(dst2_hbm.at[pl.ds(w * CPT, CPT)], dsti_v)
    plsc.subcore_barrier()

    @pl.loop(0, CPT)
    def _(i):
        pltpu.sync_copy(ones_v, acc_sh.at[dsti_v.at[i]], add=True)

    plsc.subcore_barrier()
    pltpu.sync_copy(acc_sh.at[pl.ds(s * (NP // 16), NP // 16)],
                    out_hbm.at[c].at[pl.ds(s * (NP // 16), NP // 16)])


@functools.partial(
    pl.kernel, mesh=_mesh,
    compiler_params=pltpu.CompilerParams(use_tc_tiling_on_sc=False, needs_layout_passes=False),
    out_type=jax.ShapeDtypeStruct((2, NP, H), jnp.float32),
    scratch_types=[
        pltpu.VMEM((CPT, CH), jnp.int32),
        pltpu.VMEM((CPT, CH), jnp.int32),
        pltpu.VMEM((CH, H), jnp.float32),
        pltpu.VMEM((CH, H), jnp.float32),
        pltpu.VMEM_SHARED((NP, H), jnp.float32),
        pltpu.SemaphoreType.DMA,
        pltpu.SemaphoreType.DMA,
    ],
)
def _sc_agg16(src2_hbm, dst2_hbm, y1_hbm, zrows_hbm, out_hbm,
              srci_v, dsti_v, rows_a, rows_b, acc_sh, sem_a, sem_b):
    c = lax.axis_index("c")
    s = lax.axis_index("s")
    w = _wid()
    pltpu.sync_copy(zrows_hbm, acc_sh.at[pl.ds(s * (NP // 16), NP // 16)])
    pltpu.sync_copy(src2_hbm.at[pl.ds(w * CPT, CPT)], srci_v)
    pltpu.sync_copy(dst2_hbm.at[pl.ds(w * CPT, CPT)], dsti_v)
    plsc.subcore_barrier()

    pltpu.async_copy(y1_hbm.at[srci_v.at[0]], rows_a, sem_a)

    @pl.loop(0, CPT, step=2)
    def _(g):
        pltpu.async_copy(y1_hbm.at[srci_v.at[g + 1]], rows_b, sem_b)
        pltpu.make_async_copy(y1_hbm.at[srci_v.at[g]], rows_a, sem_a).wait()
        pltpu.sync_copy(rows_a, acc_sh.at[dsti_v.at[g]], add=True)

        @pl.when(g + 2 < CPT)
        def _():
            pltpu.async_copy(y1_hbm.at[srci_v.at[g + 2]], rows_a, sem_a)

        pltpu.make_async_copy(y1_hbm.at[srci_v.at[g + 1]], rows_b, sem_b).wait()
        pltpu.sync_copy(rows_b, acc_sh.at[dsti_v.at[g + 1]], add=True)

    plsc.subcore_barrier()
    pltpu.sync_copy(acc_sh.at[pl.ds(s * (NP // 16), NP // 16)],
                    out_hbm.at[c].at[pl.ds(s * (NP // 16), NP // 16)])


@functools.partial(
    pl.kernel, mesh=_mesh,
    compiler_params=pltpu.CompilerParams(use_tc_tiling_on_sc=False, needs_layout_passes=False),
    out_type=jax.ShapeDtypeStruct((2, NP), jnp.float32),
    scratch_types=[
        pltpu.VMEM((CPT, CH), jnp.int32),
        pltpu.VMEM((CPT, CH), jnp.int32),
        pltpu.VMEM((NP,), jnp.float32),
        pltpu.VMEM((CH,), jnp.float32),
        pltpu.VMEM((CH,), jnp.float32),
        pltpu.VMEM_SHARED((NP,), jnp.float32),
        pltpu.SemaphoreType.DMA,
        pltpu.SemaphoreType.DMA,
    ],
)
def _sc_agg1(src2_hbm, dst2_hbm, y2_hbm, zeros_hbm, out_hbm,
             srci_v, dsti_v, tab_v, vals_a, vals_b, acc_sh, sem_a, sem_b):
    c = lax.axis_index("c")
    s = lax.axis_index("s")
    w = _wid()
    pltpu.sync_copy(zeros_hbm, acc_sh.at[pl.ds(s * (NP // 16), NP // 16)])
    pltpu.sync_copy(y2_hbm, tab_v)
    pltpu.sync_copy(src2_hbm.at[pl.ds(w * CPT, CPT)], srci_v)
    pltpu.sync_copy(dst2_hbm.at[pl.ds(w * CPT, CPT)], dsti_v)
    plsc.subcore_barrier()

    def _fill(i, buf):
        @pl.loop(0, CH, step=16)
        def _(j):
            idx = srci_v[i, pl.ds(j, 16)]
            buf[pl.ds(j, 16)] = plsc.load_gather(tab_v, [idx])

    _fill(0, vals_a)

    @pl.loop(0, CPT, step=2)
    def _(g):
        pltpu.async_copy(vals_a, acc_sh.at[dsti_v.at[g]], sem_a, add=True)
        _fill(g + 1, vals_b)
        pltpu.make_async_copy(vals_a, acc_sh.at[dsti_v.at[g]], sem_a).wait()
        pltpu.async_copy(vals_b, acc_sh.at[dsti_v.at[g + 1]], sem_b, add=True)

        @pl.when(g + 2 < CPT)
        def _():
            _fill(g + 2, vals_a)

        pltpu.make_async_copy(vals_b, acc_sh.at[dsti_v.at[g + 1]], sem_b).wait()

    plsc.subcore_barrier()
    pltpu.sync_copy(acc_sh.at[pl.ds(s * (NP // 16), NP // 16)],
                    out_hbm.at[c].at[pl.ds(s * (NP // 16), NP // 16)])


def _tc_mm1_body(x_ref, w_ref, o_ref):
    o_ref[...] = jnp.dot(x_ref[...], w_ref[...],
                         preferred_element_type=jnp.float32)


def _tc_scale_body(h0_ref, h1_ref, xw_ref, y1_ref, dinv_ref):
    deg = h0_ref[...] + h1_ref[...] + 1.0
    dinv = lax.rsqrt(deg)
    dinv_ref[...] = dinv
    y1_ref[...] = xw_ref[...] * dinv


def _tc_layer2_body(a0_ref, a1_ref, y1_ref, dinv_ref, b1_ref, w2_ref, o_ref):
    pre = (a0_ref[...] + a1_ref[...] + y1_ref[...]) * dinv_ref[...] + b1_ref[...]
    h = jnp.maximum(pre, 0.0)
    z = jnp.dot(h, w2_ref[...], preferred_element_type=jnp.float32)
    o_ref[0:N, :] = z * dinv_ref[...]
    o_ref[N:NP, :] = jnp.zeros((NP - N, 1), jnp.float32)


def _tc_final_body(p0_ref, p1_ref, y2_ref, dinv_ref, b2_ref, o_ref):
    o_ref[...] = dinv_ref[...] * (p0_ref[...] + p1_ref[...] + y2_ref[...]) \
        + b2_ref[...]


def kernel(x, edge_index, W1, b1, W2, b2):
    f32 = jnp.float32
    npad = EP - E
    pad_src = jnp.zeros((npad,), jnp.int32)
    pad_dst = N + (jnp.arange(npad, dtype=jnp.int32) % (NP - N))
    src2 = jnp.concatenate([edge_index[0], pad_src]).reshape(EP // CH, CH)
    dst2 = jnp.concatenate([edge_index[1], pad_dst]).reshape(EP // CH, CH)
    zeros_v = jnp.zeros((NP // 16,), f32)
    zeros_r = jnp.zeros((NP // 16, H), f32)
    ones_v = jnp.ones((CH,), f32)

    xw = pl.pallas_call(
        _tc_mm1_body,
        out_shape=jax.ShapeDtypeStruct((N, H), f32),
    )(x, W1)

    hist = _sc_hist(dst2, zeros_v, ones_v)
    h0 = hist[0, :N].reshape(N, 1)
    h1 = hist[1, :N].reshape(N, 1)

    y1, dinv = pl.pallas_call(
        _tc_scale_body,
        out_shape=(jax.ShapeDtypeStruct((N, H), f32),
                   jax.ShapeDtypeStruct((N, 1), f32)),
    )(h0, h1, xw)

    agg = _sc_agg16(src2, dst2, y1, zeros_r)

    y2p = pl.pallas_call(
        _tc_layer2_body,
        out_shape=jax.ShapeDtypeStruct((NP, 1), f32),
    )(agg[0, :N], agg[1, :N], y1, dinv, b1.reshape(1, H), W2)

    p = _sc_agg1(src2, dst2, y2p.reshape(NP), zeros_v)

    out = pl.pallas_call(
        _tc_final_body,
        out_shape=jax.ShapeDtypeStruct((N, 1), f32),
    )(p[0, :N].reshape(N, 1), p[1, :N].reshape(N, 1), y2p[:N],
      dinv, b2.reshape(1, 1))

    return out.reshape(-1)


# speedup vs baseline: 51.9870x; 1.0306x over previous
import functools

import jax
import jax.numpy as jnp
from jax import lax
from jax.experimental import pallas as pl
from jax.experimental.pallas import tpu as pltpu
from jax.experimental.pallas import tpu_sc as plsc

N = 10000
E = 320000
D = 128
H = 16
NP = 10240
NW = 32
CH = 128
CPT = 80
EP = NW * CPT * CH

_mesh = plsc.VectorSubcoreMesh(core_axis_name="c", subcore_axis_name="s")


def _wid():
    return lax.axis_index("s") * 2 + lax.axis_index("c")


@functools.partial(
    pl.kernel, mesh=_mesh,
    compiler_params=pltpu.CompilerParams(use_tc_tiling_on_sc=False, needs_layout_passes=False),
    out_type=jax.ShapeDtypeStruct((2, NP), jnp.float32),
    scratch_types=[
        pltpu.VMEM((CPT, CH), jnp.int32),
        pltpu.VMEM((CH,), jnp.float32),
        pltpu.VMEM_SHARED((NP,), jnp.float32),
        pltpu.SemaphoreType.DMA,
    ],
)
def _sc_hist(dst2_hbm, zeros_hbm, ones_hbm, out_hbm, dsti_v, ones_v, acc_sh,
             sem):
    c = lax.axis_index("c")
    s = lax.axis_index("s")
    w = _wid()
    pltpu.sync_copy(zeros_hbm, acc_sh.at[pl.ds(s * (NP // 16), NP // 16)])
    pltpu.sync_copy(ones_hbm, ones_v)
    pltpu.sync_copy(dst2_hbm.at[pl.ds(w * CPT, CPT)], dsti_v)
    plsc.subcore_barrier()

    @pl.loop(0, CPT)
    def _(i):
        pltpu.async_copy(ones_v, acc_sh.at[dsti_v.at[i]], sem, add=True)

    @pl.loop(0, CPT)
    def _(i):
        pltpu.make_async_copy(ones_v, acc_sh.at[dsti_v.at[i]], sem).wait()

    plsc.subcore_barrier()
    pltpu.sync_copy(acc_sh.at[pl.ds(s * (NP // 16), NP // 16)],
                    out_hbm.at[c].at[pl.ds(s * (NP // 16), NP // 16)])


@functools.partial(
    pl.kernel, mesh=_mesh,
    compiler_params=pltpu.CompilerParams(use_tc_tiling_on_sc=False, needs_layout_passes=False),
    out_type=jax.ShapeDtypeStruct((2, NP, H), jnp.float32),
    scratch_types=[
        pltpu.VMEM((CPT, CH), jnp.int32),
        pltpu.VMEM((CPT, CH), jnp.int32),
        [pltpu.VMEM((CH, H), jnp.float32)] * 4,
        [pltpu.SemaphoreType.DMA] * 4,
        [pltpu.SemaphoreType.DMA] * 4,
        pltpu.VMEM_SHARED((NP, H), jnp.float32),
    ],
)
def _sc_agg16(src2_hbm, dst2_hbm, y1_hbm, zrows_hbm, out_hbm,
              srci_v, dsti_v, rows, gsem, ssem, acc_sh):
    c = lax.axis_index("c")
    s = lax.axis_index("s")
    w = _wid()
    pltpu.sync_copy(zrows_hbm, acc_sh.at[pl.ds(s * (NP // 16), NP // 16)])
    pltpu.sync_copy(src2_hbm.at[pl.ds(w * CPT, CPT)], srci_v)
    pltpu.sync_copy(dst2_hbm.at[pl.ds(w * CPT, CPT)], dsti_v)
    plsc.subcore_barrier()

    def _gather(i, b):
        pltpu.async_copy(y1_hbm.at[srci_v.at[i]], rows[b], gsem[b])

    def _wait_gather(i, b):
        pltpu.make_async_copy(y1_hbm.at[srci_v.at[i]], rows[b], gsem[b]).wait()

    def _scatter(i, b):
        pltpu.async_copy(rows[b], acc_sh.at[dsti_v.at[i]], ssem[b], add=True)

    def _wait_scatter(i, b):
        pltpu.make_async_copy(rows[b], acc_sh.at[dsti_v.at[i]], ssem[b]).wait()

    _gather(0, 0)
    _gather(1, 1)

    @pl.loop(0, CPT, step=4)
    def _(g):
        for b in range(4):
            j = g + b
            _wait_gather(j, b)
            _scatter(j, b)
            i = j + 2
            bi = (b + 2) % 4

            @pl.when(i < CPT)
            def _():
                @pl.when(j >= 2)
                def _():
                    _wait_scatter(j - 2, bi)

                _gather(i, bi)

    _wait_scatter(CPT - 2, (CPT - 2) % 4)
    _wait_scatter(CPT - 1, (CPT - 1) % 4)
    plsc.subcore_barrier()
    pltpu.sync_copy(acc_sh.at[pl.ds(s * (NP // 16), NP // 16)],
                    out_hbm.at[c].at[pl.ds(s * (NP // 16), NP // 16)])


@functools.partial(
    pl.kernel, mesh=_mesh,
    compiler_params=pltpu.CompilerParams(use_tc_tiling_on_sc=False, needs_layout_passes=False),
    out_type=jax.ShapeDtypeStruct((2, NP), jnp.float32),
    scratch_types=[
        pltpu.VMEM((CPT, CH), jnp.int32),
        pltpu.VMEM((CPT, CH), jnp.int32),
        pltpu.VMEM((NP,), jnp.float32),
        pltpu.VMEM((CH,), jnp.float32),
        pltpu.VMEM((CH,), jnp.float32),
        pltpu.VMEM_SHARED((NP,), jnp.float32),
        pltpu.SemaphoreType.DMA,
        pltpu.SemaphoreType.DMA,
    ],
)
def _sc_agg1(src2_hbm, dst2_hbm, y2_hbm, zeros_hbm, out_hbm,
             srci_v, dsti_v, tab_v, vals_a, vals_b, acc_sh, sem_a, sem_b):
    c = lax.axis_index("c")
    s = lax.axis_index("s")
    w = _wid()
    pltpu.sync_copy(zeros_hbm, acc_sh.at[pl.ds(s * (NP // 16), NP // 16)])
    pltpu.sync_copy(y2_hbm, tab_v)
    pltpu.sync_copy(src2_hbm.at[pl.ds(w * CPT, CPT)], srci_v)
    pltpu.sync_copy(dst2_hbm.at[pl.ds(w * CPT, CPT)], dsti_v)
    plsc.subcore_barrier()

    def _fill(i, buf):
        @pl.loop(0, CH, step=16)
        def _(j):
            idx = srci_v[i, pl.ds(j, 16)]
            buf[pl.ds(j, 16)] = plsc.load_gather(tab_v, [idx])

    _fill(0, vals_a)

    @pl.loop(0, CPT, step=2)
    def _(g):
        pltpu.async_copy(vals_a, acc_sh.at[dsti_v.at[g]], sem_a, add=True)
        _fill(g + 1, vals_b)
        pltpu.make_async_copy(vals_a, acc_sh.at[dsti_v.at[g]], sem_a).wait()
        pltpu.async_copy(vals_b, acc_sh.at[dsti_v.at[g + 1]], sem_b, add=True)

        @pl.when(g + 2 < CPT)
        def _():
            _fill(g + 2, vals_a)

        pltpu.make_async_copy(vals_b, acc_sh.at[dsti_v.at[g + 1]], sem_b).wait()

    plsc.subcore_barrier()
    pltpu.sync_copy(acc_sh.at[pl.ds(s * (NP // 16), NP // 16)],
                    out_hbm.at[c].at[pl.ds(s * (NP // 16), NP // 16)])


def _tc_mm1_body(x_ref, w_ref, o_ref):
    o_ref[...] = jnp.dot(x_ref[...], w_ref[...],
                         preferred_element_type=jnp.float32)


def _tc_scale_body(h0_ref, h1_ref, xw_ref, y1_ref, dinv_ref):
    deg = h0_ref[...] + h1_ref[...] + 1.0
    dinv = lax.rsqrt(deg)
    dinv_ref[...] = dinv
    y1_ref[...] = xw_ref[...] * dinv


def _tc_layer2_body(a0_ref, a1_ref, y1_ref, dinv_ref, b1_ref, w2_ref, o_ref):
    pre = (a0_ref[...] + a1_ref[...] + y1_ref[...]) * dinv_ref[...] + b1_ref[...]
    h = jnp.maximum(pre, 0.0)
    z = jnp.dot(h, w2_ref[...], preferred_element_type=jnp.float32)
    o_ref[0:N, :] = z * dinv_ref[...]
    o_ref[N:NP, :] = jnp.zeros((NP - N, 1), jnp.float32)


def _tc_final_body(p0_ref, p1_ref, y2_ref, dinv_ref, b2_ref, o_ref):
    o_ref[...] = dinv_ref[...] * (p0_ref[...] + p1_ref[...] + y2_ref[...]) \
        + b2_ref[...]


def kernel(x, edge_index, W1, b1, W2, b2):
    f32 = jnp.float32
    npad = EP - E
    pad_src = jnp.zeros((npad,), jnp.int32)
    pad_dst = N + (jnp.arange(npad, dtype=jnp.int32) % (NP - N))
    src2 = jnp.concatenate([edge_index[0], pad_src]).reshape(EP // CH, CH)
    dst2 = jnp.concatenate([edge_index[1], pad_dst]).reshape(EP // CH, CH)
    zeros_v = jnp.zeros((NP // 16,), f32)
    zeros_r = jnp.zeros((NP // 16, H), f32)
    ones_v = jnp.ones((CH,), f32)

    xw = pl.pallas_call(
        _tc_mm1_body,
        out_shape=jax.ShapeDtypeStruct((N, H), f32),
    )(x, W1)

    hist = _sc_hist(dst2, zeros_v, ones_v)
    h0 = hist[0, :N].reshape(N, 1)
    h1 = hist[1, :N].reshape(N, 1)

    y1, dinv = pl.pallas_call(
        _tc_scale_body,
        out_shape=(jax.ShapeDtypeStruct((N, H), f32),
                   jax.ShapeDtypeStruct((N, 1), f32)),
    )(h0, h1, xw)

    agg = _sc_agg16(src2, dst2, y1, zeros_r)

    y2p = pl.pallas_call(
        _tc_layer2_body,
        out_shape=jax.ShapeDtypeStruct((NP, 1), f32),
    )(agg[0, :N], agg[1, :N], y1, dinv, b1.reshape(1, H), W2)

    p = _sc_agg1(src2, dst2, y2p.reshape(NP), zeros_v)

    out = pl.pallas_call(
        _tc_final_body,
        out_shape=jax.ShapeDtypeStruct((N, 1), f32),
    )(p[0, :N].reshape(N, 1), p[1, :N].reshape(N, 1), y2p[:N],
      dinv, b2.reshape(1, 1))

    return out.reshape(-1)


# speedup vs baseline: 63.6850x; 1.2250x over previous
import functools

import jax
import jax.numpy as jnp
from jax import lax
from jax.experimental import pallas as pl
from jax.experimental.pallas import tpu as pltpu
from jax.experimental.pallas import tpu_sc as plsc

N = 10000
E = 320000
D = 128
H = 16
NP = 10240
NW = 32
CH = 128
CPT = 80
EP = NW * CPT * CH

_mesh = plsc.VectorSubcoreMesh(core_axis_name="c", subcore_axis_name="s")


def _wid():
    return lax.axis_index("s") * 2 + lax.axis_index("c")


@functools.partial(
    pl.kernel, mesh=_mesh,
    compiler_params=pltpu.CompilerParams(use_tc_tiling_on_sc=False, needs_layout_passes=False),
    out_type=jax.ShapeDtypeStruct((2, NP), jnp.float32),
    scratch_types=[
        pltpu.VMEM((CPT, CH), jnp.int32),
        pltpu.VMEM((CH,), jnp.float32),
        pltpu.VMEM_SHARED((NP,), jnp.float32),
        pltpu.SemaphoreType.DMA,
    ],
)
def _sc_hist(dst2_hbm, zeros_hbm, ones_hbm, out_hbm, dsti_v, ones_v, acc_sh,
             sem):
    c = lax.axis_index("c")
    s = lax.axis_index("s")
    w = _wid()
    pltpu.sync_copy(zeros_hbm, acc_sh.at[pl.ds(s * (NP // 16), NP // 16)])
    pltpu.sync_copy(ones_hbm, ones_v)
    pltpu.sync_copy(dst2_hbm.at[pl.ds(w * CPT, CPT)], dsti_v)
    plsc.subcore_barrier()

    @pl.loop(0, CPT)
    def _(i):
        pltpu.async_copy(ones_v, acc_sh.at[dsti_v.at[i]], sem, add=True)

    @pl.loop(0, CPT)
    def _(i):
        pltpu.make_async_copy(ones_v, acc_sh.at[dsti_v.at[i]], sem).wait()

    plsc.subcore_barrier()
    pltpu.sync_copy(acc_sh.at[pl.ds(s * (NP // 16), NP // 16)],
                    out_hbm.at[c].at[pl.ds(s * (NP // 16), NP // 16)])


@functools.partial(
    pl.kernel, mesh=_mesh,
    compiler_params=pltpu.CompilerParams(use_tc_tiling_on_sc=False, needs_layout_passes=False),
    out_type=jax.ShapeDtypeStruct((2, NP, H), jnp.float32),
    scratch_types=[
        pltpu.VMEM((CPT, CH), jnp.int32),
        pltpu.VMEM((CPT, CH), jnp.int32),
        [pltpu.VMEM((CH, H), jnp.float32)] * 4,
        [pltpu.SemaphoreType.DMA] * 4,
        [pltpu.SemaphoreType.DMA] * 4,
        pltpu.VMEM_SHARED((NP, H), jnp.float32),
        pltpu.VMEM_SHARED((NP, H), jnp.float32),
    ],
)
def _sc_agg16(src2_hbm, dst2_hbm, y1_hbm, zrows_hbm, out_hbm,
              srci_v, dsti_v, rows, gsem, ssem, acc_sh, tab_sh):
    c = lax.axis_index("c")
    s = lax.axis_index("s")
    w = _wid()
    pltpu.sync_copy(zrows_hbm, acc_sh.at[pl.ds(s * (NP // 16), NP // 16)])
    pltpu.sync_copy(y1_hbm.at[pl.ds(s * (NP // 16), NP // 16)],
                    tab_sh.at[pl.ds(s * (NP // 16), NP // 16)])
    pltpu.sync_copy(src2_hbm.at[pl.ds(w * CPT, CPT)], srci_v)
    pltpu.sync_copy(dst2_hbm.at[pl.ds(w * CPT, CPT)], dsti_v)
    plsc.subcore_barrier()

    def _gather(i, b):
        pltpu.async_copy(tab_sh.at[srci_v.at[i]], rows[b], gsem[b])

    def _wait_gather(i, b):
        pltpu.make_async_copy(tab_sh.at[srci_v.at[i]], rows[b], gsem[b]).wait()

    def _scatter(i, b):
        pltpu.async_copy(rows[b], acc_sh.at[dsti_v.at[i]], ssem[b], add=True)

    def _wait_scatter(i, b):
        pltpu.make_async_copy(rows[b], acc_sh.at[dsti_v.at[i]], ssem[b]).wait()

    _gather(0, 0)
    _gather(1, 1)

    @pl.loop(0, CPT, step=4)
    def _(g):
        for b in range(4):
            j = g + b
            _wait_gather(j, b)
            _scatter(j, b)
            i = j + 2
            bi = (b + 2) % 4

            @pl.when(i < CPT)
            def _():
                @pl.when(j >= 2)
                def _():
                    _wait_scatter(j - 2, bi)

                _gather(i, bi)

    _wait_scatter(CPT - 2, (CPT - 2) % 4)
    _wait_scatter(CPT - 1, (CPT - 1) % 4)
    plsc.subcore_barrier()
    pltpu.sync_copy(acc_sh.at[pl.ds(s * (NP // 16), NP // 16)],
                    out_hbm.at[c].at[pl.ds(s * (NP // 16), NP // 16)])


@functools.partial(
    pl.kernel, mesh=_mesh,
    compiler_params=pltpu.CompilerParams(use_tc_tiling_on_sc=False, needs_layout_passes=False),
    out_type=jax.ShapeDtypeStruct((2, NP), jnp.float32),
    scratch_types=[
        pltpu.VMEM((CPT, CH), jnp.int32),
        pltpu.VMEM((CPT, CH), jnp.int32),
        pltpu.VMEM((NP,), jnp.float32),
        pltpu.VMEM((CH,), jnp.float32),
        pltpu.VMEM((CH,), jnp.float32),
        pltpu.VMEM_SHARED((NP,), jnp.float32),
        pltpu.SemaphoreType.DMA,
        pltpu.SemaphoreType.DMA,
    ],
)
def _sc_agg1(src2_hbm, dst2_hbm, y2_hbm, zeros_hbm, out_hbm,
             srci_v, dsti_v, tab_v, vals_a, vals_b, acc_sh, sem_a, sem_b):
    c = lax.axis_index("c")
    s = lax.axis_index("s")
    w = _wid()
    pltpu.sync_copy(zeros_hbm, acc_sh.at[pl.ds(s * (NP // 16), NP // 16)])
    pltpu.sync_copy(y2_hbm, tab_v)
    pltpu.sync_copy(src2_hbm.at[pl.ds(w * CPT, CPT)], srci_v)
    pltpu.sync_copy(dst2_hbm.at[pl.ds(w * CPT, CPT)], dsti_v)
    plsc.subcore_barrier()

    def _fill(i, buf):
        @pl.loop(0, CH, step=16)
        def _(j):
            idx = srci_v[i, pl.ds(j, 16)]
            buf[pl.ds(j, 16)] = plsc.load_gather(tab_v, [idx])

    _fill(0, vals_a)

    @pl.loop(0, CPT, step=2)
    def _(g):
        pltpu.async_copy(vals_a, acc_sh.at[dsti_v.at[g]], sem_a, add=True)
        _fill(g + 1, vals_b)
        pltpu.make_async_copy(vals_a, acc_sh.at[dsti_v.at[g]], sem_a).wait()
        pltpu.async_copy(vals_b, acc_sh.at[dsti_v.at[g + 1]], sem_b, add=True)

        @pl.when(g + 2 < CPT)
        def _():
            _fill(g + 2, vals_a)

        pltpu.make_async_copy(vals_b, acc_sh.at[dsti_v.at[g + 1]], sem_b).wait()

    plsc.subcore_barrier()
    pltpu.sync_copy(acc_sh.at[pl.ds(s * (NP // 16), NP // 16)],
                    out_hbm.at[c].at[pl.ds(s * (NP // 16), NP // 16)])


def _tc_mm1_body(x_ref, w_ref, o_ref):
    o_ref[...] = jnp.dot(x_ref[...], w_ref[...],
                         preferred_element_type=jnp.float32)


def _tc_scale_body(h0_ref, h1_ref, xw_ref, y1_ref, dinv_ref):
    deg = h0_ref[...] + h1_ref[...] + 1.0
    dinv = lax.rsqrt(deg)
    dinv_ref[...] = dinv
    y1_ref[0:N, :] = xw_ref[...] * dinv
    y1_ref[N:NP, :] = jnp.zeros((NP - N, H), jnp.float32)


def _tc_layer2_body(a0_ref, a1_ref, y1_ref, dinv_ref, b1_ref, w2_ref, o_ref):
    pre = (a0_ref[...] + a1_ref[...] + y1_ref[...]) * dinv_ref[...] + b1_ref[...]
    h = jnp.maximum(pre, 0.0)
    z = jnp.dot(h, w2_ref[...], preferred_element_type=jnp.float32)
    o_ref[0:N, :] = z * dinv_ref[...]
    o_ref[N:NP, :] = jnp.zeros((NP - N, 1), jnp.float32)


def _tc_final_body(p0_ref, p1_ref, y2_ref, dinv_ref, b2_ref, o_ref):
    o_ref[...] = dinv_ref[...] * (p0_ref[...] + p1_ref[...] + y2_ref[...]) \
        + b2_ref[...]


def kernel(x, edge_index, W1, b1, W2, b2):
    f32 = jnp.float32
    npad = EP - E
    pad_src = jnp.zeros((npad,), jnp.int32)
    pad_dst = N + (jnp.arange(npad, dtype=jnp.int32) % (NP - N))
    src2 = jnp.concatenate([edge_index[0], pad_src]).reshape(EP // CH, CH)
    dst2 = jnp.concatenate([edge_index[1], pad_dst]).reshape(EP // CH, CH)
    zeros_v = jnp.zeros((NP // 16,), f32)
    zeros_r = jnp.zeros((NP // 16, H), f32)
    ones_v = jnp.ones((CH,), f32)

    xw = pl.pallas_call(
        _tc_mm1_body,
        out_shape=jax.ShapeDtypeStruct((N, H), f32),
    )(x, W1)

    hist = _sc_hist(dst2, zeros_v, ones_v)
    h0 = hist[0, :N].reshape(N, 1)
    h1 = hist[1, :N].reshape(N, 1)

    y1p, dinv = pl.pallas_call(
        _tc_scale_body,
        out_shape=(jax.ShapeDtypeStruct((NP, H), f32),
                   jax.ShapeDtypeStruct((N, 1), f32)),
    )(h0, h1, xw)
    y1 = y1p[:N]

    agg = _sc_agg16(src2, dst2, y1p, zeros_r)

    y2p = pl.pallas_call(
        _tc_layer2_body,
        out_shape=jax.ShapeDtypeStruct((NP, 1), f32),
    )(agg[0, :N], agg[1, :N], y1, dinv, b1.reshape(1, H), W2)

    p = _sc_agg1(src2, dst2, y2p.reshape(NP), zeros_v)

    out = pl.pallas_call(
        _tc_final_body,
        out_shape=jax.ShapeDtypeStruct((N, 1), f32),
    )(p[0, :N].reshape(N, 1), p[1, :N].reshape(N, 1), y2p[:N],
      dinv, b2.reshape(1, 1))

    return out.reshape(-1)
